# Initial kernel scaffold; baseline (speedup 1.0000x reference)
import jax
import jax.numpy as jnp
from jax.experimental import pallas as pl


def kernel(x, edgeIndex, edgeAttribute, W, b):
    raise NotImplementedError("write your pallas kernel here")


# speedup vs baseline: 12.1719x; 12.1719x over previous
import functools

import jax
import jax.numpy as jnp
from jax import lax
from jax.experimental import pallas as pl
from jax.experimental.pallas import tpu as pltpu
from jax.experimental.pallas import tpu_sc as plsc

N = 10000
E = 320000
C = 128
NC = 2
NS = 16
NW = NC * NS
NPAD = 10240
EPT = E // NW
CHUNK = 80
NCH = EPT // CHUNK
RPT = NPAD // NS

def _sc_deg_body(row_hbm, ea_hbm, out_hbm, rbig_v, wbig_v, ridx_v, wsm_v,
                 zb_v, acc_sh):
    cid = lax.axis_index("c")
    sid = lax.axis_index("s")
    wid = cid * NS + sid
    pltpu.sync_copy(row_hbm.at[pl.ds(wid * EPT, EPT)], rbig_v)
    pltpu.sync_copy(ea_hbm.at[pl.ds(wid * EPT, EPT)], wbig_v)

    def zero_zb(i, carry):
        zb_v[pl.ds(i * 16, 16)] = jnp.zeros((16,), jnp.float32)
        return carry

    lax.fori_loop(0, RPT // 16, zero_zb, 0)
    pltpu.sync_copy(zb_v, acc_sh.at[pl.ds(sid * RPT, RPT)])
    plsc.subcore_barrier()

    def chunk(i, carry):
        def cp(g, c2):
            ridx_v[pl.ds(g * 16, 16)] = rbig_v[pl.ds(i * CHUNK + g * 16, 16)]
            wsm_v[pl.ds(g * 16, 16)] = wbig_v[pl.ds(i * CHUNK + g * 16, 16)]
            return c2

        lax.fori_loop(0, CHUNK // 16, cp, 0)
        pltpu.sync_copy(wsm_v, acc_sh.at[ridx_v], add=True)
        return carry

    lax.fori_loop(0, NCH, chunk, 0)
    plsc.subcore_barrier()
    pltpu.sync_copy(acc_sh.at[pl.ds(sid * RPT, RPT)],
                    out_hbm.at[pl.ds(cid * NPAD + sid * RPT, RPT)])


def _sc_prop_body(u_hbm, row_hbm, col_hbm, ea_hbm, out_hbm,
                  rbig_v, cbig_v, wbig_v, ridx_v, cidx_v, rows_v,
                  acc_sh, sem):
    cid = lax.axis_index("c")
    sid = lax.axis_index("s")
    wid = cid * NS + sid
    pltpu.sync_copy(row_hbm.at[pl.ds(wid * EPT, EPT)], rbig_v)
    pltpu.sync_copy(col_hbm.at[pl.ds(wid * EPT, EPT)], cbig_v)
    pltpu.sync_copy(ea_hbm.at[pl.ds(wid * EPT, EPT)], wbig_v)

    def zero_rows(i, carry):
        for c8 in range(C // 16):
            rows_v[i, pl.ds(c8 * 16, 16)] = jnp.zeros((16,), jnp.float32)
        return carry

    lax.fori_loop(0, CHUNK, zero_rows, 0)
    for r in range(RPT // CHUNK):
        pltpu.sync_copy(rows_v, acc_sh.at[pl.ds(sid * RPT + r * CHUNK, CHUNK)])
    plsc.subcore_barrier()

    def chunk(i, carry):
        def cp(g, c2):
            ridx_v[pl.ds(g * 16, 16)] = rbig_v[pl.ds(i * CHUNK + g * 16, 16)]
            cidx_v[pl.ds(g * 16, 16)] = cbig_v[pl.ds(i * CHUNK + g * 16, 16)]
            return c2

        lax.fori_loop(0, CHUNK // 16, cp, 0)
        pltpu.async_copy(u_hbm.at[ridx_v], rows_v, sem).wait()

        def grp(g, c2):
            w16 = wbig_v[pl.ds(i * CHUNK + g * 16, 16)]
            for l in range(16):
                wl = w16[l]
                e = g * 16 + l
                for c8 in range(C // 16):
                    rows_v[e, pl.ds(c8 * 16, 16)] = (
                        rows_v[e, pl.ds(c8 * 16, 16)] * wl)
            return c2

        lax.fori_loop(0, CHUNK // 16, grp, 0)
        pltpu.sync_copy(rows_v, acc_sh.at[cidx_v], add=True)
        return carry

    lax.fori_loop(0, NCH, chunk, 0)
    plsc.subcore_barrier()
    for r in range(RPT // CHUNK):
        pltpu.sync_copy(acc_sh.at[pl.ds(sid * RPT + r * CHUNK, CHUNK)],
                        out_hbm.at[cid].at[pl.ds(sid * RPT + r * CHUNK, CHUNK)])


@functools.cache
def _sc_kernels():
    mesh = plsc.VectorSubcoreMesh(
        core_axis_name="c", subcore_axis_name="s",
        num_cores=NC, num_subcores=NS)
    sc_deg = pl.kernel(
        _sc_deg_body,
        out_type=jax.ShapeDtypeStruct((NC * NPAD,), jnp.float32),
        mesh=mesh,
        scratch_types=[
            pltpu.VMEM((EPT,), jnp.int32),
            pltpu.VMEM((EPT,), jnp.float32),
            pltpu.VMEM((CHUNK,), jnp.int32),
            pltpu.VMEM((CHUNK,), jnp.float32),
            pltpu.VMEM((RPT,), jnp.float32),
            pltpu.VMEM_SHARED((NPAD,), jnp.float32),
        ],
    )
    sc_prop = pl.kernel(
        _sc_prop_body,
        out_type=jax.ShapeDtypeStruct((NC, NPAD, C), jnp.float32),
        mesh=mesh,
        scratch_types=[
            pltpu.VMEM((EPT,), jnp.int32),
            pltpu.VMEM((EPT,), jnp.int32),
            pltpu.VMEM((EPT,), jnp.float32),
            pltpu.VMEM((CHUNK,), jnp.int32),
            pltpu.VMEM((CHUNK,), jnp.int32),
            pltpu.VMEM((CHUNK, C), jnp.float32),
            pltpu.VMEM_SHARED((NPAD, C), jnp.float32),
            pltpu.SemaphoreType.DMA,
        ],
    )
    return sc_deg, sc_prop


BS = 1000
GRID = N // BS


def _tc_pre_body(dp_ref, x_ref, dinv_ref, u0_ref):
    deg = dp_ref[0] + dp_ref[1]
    pos = deg > 0.0
    dinv = jnp.where(pos, lax.rsqrt(jnp.where(pos, deg, 1.0)), 0.0)
    dinv_ref[...] = dinv
    u0_ref[...] = dinv * x_ref[...]


_tc_pre = pl.pallas_call(
    _tc_pre_body,
    grid=(GRID,),
    in_specs=[
        pl.BlockSpec((NC, BS, 1), lambda i: (0, i, 0)),
        pl.BlockSpec((BS, C), lambda i: (i, 0)),
    ],
    out_specs=[
        pl.BlockSpec((BS, 1), lambda i: (i, 0)),
        pl.BlockSpec((BS, C), lambda i: (i, 0)),
    ],
    out_shape=[
        jax.ShapeDtypeStruct((N, 1), jnp.float32),
        jax.ShapeDtypeStruct((N, C), jnp.float32),
    ],
)


def _tc_mid_body(yp_ref, dinv_ref, x_ref, w0_ref, w1_ref, u1_ref, part_ref):
    y = yp_ref[0] + yp_ref[1]
    dinv = dinv_ref[...]
    tx1 = -dinv * y
    u1_ref[...] = dinv * tx1
    part_ref[...] = (
        jnp.dot(x_ref[...], w0_ref[...], preferred_element_type=jnp.float32)
        + jnp.dot(tx1, w1_ref[...], preferred_element_type=jnp.float32))


_tc_mid = pl.pallas_call(
    _tc_mid_body,
    grid=(GRID,),
    in_specs=[
        pl.BlockSpec((NC, BS, C), lambda i: (0, i, 0)),
        pl.BlockSpec((BS, 1), lambda i: (i, 0)),
        pl.BlockSpec((BS, C), lambda i: (i, 0)),
        pl.BlockSpec((C, C), lambda i: (0, 0)),
        pl.BlockSpec((C, C), lambda i: (0, 0)),
    ],
    out_specs=[
        pl.BlockSpec((BS, C), lambda i: (i, 0)),
        pl.BlockSpec((BS, C), lambda i: (i, 0)),
    ],
    out_shape=[
        jax.ShapeDtypeStruct((N, C), jnp.float32),
        jax.ShapeDtypeStruct((N, C), jnp.float32),
    ],
)


def _tc_post_body(yp_ref, dinv_ref, x_ref, part_ref, w2_ref, b_ref, o_ref):
    y = yp_ref[0] + yp_ref[1]
    tx2 = -2.0 * dinv_ref[...] * y - x_ref[...]
    o = (part_ref[...]
         + jnp.dot(tx2, w2_ref[...], preferred_element_type=jnp.float32)
         + b_ref[...])
    o_ref[...] = jnp.where(o >= 0.0, o, 0.01 * o)


_tc_post = pl.pallas_call(
    _tc_post_body,
    grid=(GRID,),
    in_specs=[
        pl.BlockSpec((NC, BS, C), lambda i: (0, i, 0)),
        pl.BlockSpec((BS, 1), lambda i: (i, 0)),
        pl.BlockSpec((BS, C), lambda i: (i, 0)),
        pl.BlockSpec((BS, C), lambda i: (i, 0)),
        pl.BlockSpec((C, C), lambda i: (0, 0)),
        pl.BlockSpec((1, C), lambda i: (0, 0)),
    ],
    out_specs=pl.BlockSpec((BS, C), lambda i: (i, 0)),
    out_shape=jax.ShapeDtypeStruct((N, C), jnp.float32),
)


def kernel(x, edgeIndex, edgeAttribute, W, b):
    row = edgeIndex[0]
    col = edgeIndex[1]
    _sc_deg, _sc_prop = _sc_kernels()

    deg_part = _sc_deg(row, edgeAttribute)
    dp = deg_part.reshape(NC, NPAD, 1)
    dinv, u0 = _tc_pre(dp, x)
    y1p = _sc_prop(u0, row, col, edgeAttribute)
    u1, part = _tc_mid(y1p, dinv, x, W[0], W[1])
    y2p = _sc_prop(u1, row, col, edgeAttribute)
    out = _tc_post(y2p, dinv, x, part, W[2], b.reshape(1, C))
    return out
